# Initial kernel scaffold; baseline (speedup 1.0000x reference)
import jax
import jax.numpy as jnp
from jax.experimental import pallas as pl


def kernel(log_probs, input_lens, labels, A_scores):
    raise NotImplementedError("write your pallas kernel here")


# speedup vs baseline: 19.7455x; 19.7455x over previous
import functools

import jax
import jax.numpy as jnp
from jax import lax
from jax.experimental import pallas as pl
from jax.experimental.pallas import tpu as pltpu
from jax.experimental.pallas import tpu_sc as plsc

L = 16
NCLASS = 18
LANES = 16
TABLE_PAD = 384


def _make_sc_call(BS, T, C, n_workers):
    seq_per_w = BS // n_workers
    DCHUNK = 2048
    n_chunks = T // DCHUNK
    LPAD = T + LANES
    mesh = plsc.VectorSubcoreMesh(core_axis_name="c", subcore_axis_name="s")

    @functools.partial(
        pl.kernel,
        out_type=jax.ShapeDtypeStruct((n_workers * LANES,), jnp.float32),
        mesh=mesh,
        scratch_types=[
            pltpu.VMEM((seq_per_w * LPAD,), jnp.int32),
            pltpu.VMEM((DCHUNK * C,), jnp.float32),
            pltpu.VMEM((DCHUNK * C,), jnp.float32),
            pltpu.VMEM((TABLE_PAD,), jnp.float32),
            pltpu.VMEM((LANES,), jnp.float32),
            pltpu.SemaphoreType.DMA,
            pltpu.SemaphoreType.DMA,
        ],
        compiler_params=pltpu.CompilerParams(needs_layout_passes=False),
    )
    def sc_call(lp_hbm, lab_hbm, tab_hbm, out_hbm,
                lab_v, ch0, ch1, tab_v, acc_v, sem0, sem1):
        nc = mesh.num_cores
        wid = lax.axis_index("s") * nc + lax.axis_index("c")
        iota = lax.iota(jnp.int32, LANES)

        pltpu.sync_copy(tab_hbm, tab_v)
        for s in range(seq_per_w):
            seq = wid * seq_per_w + s
            pltpu.sync_copy(lab_hbm.at[pl.ds(seq * T, T)],
                            lab_v.at[pl.ds(s * LPAD, T)])
            lab_v[pl.ds(s * LPAD + T, LANES)] = jnp.full(
                (LANES,), NCLASS, jnp.int32)

        bufs = (ch0, ch1)
        sems = (sem0, sem1)
        steps = [(s, c) for s in range(seq_per_w) for c in range(n_chunks)]

        def start_dma(step_i):
            s, c = steps[step_i]
            seq = wid * seq_per_w + s
            buf = bufs[step_i % 2]
            return pltpu.async_copy(
                lp_hbm.at[pl.ds((seq * T + c * DCHUNK) * C, DCHUNK * C)],
                buf, sems[step_i % 2])

        acc = jnp.zeros((LANES,), jnp.float32)
        pending = start_dma(0)
        for step_i, (s, c) in enumerate(steps):
            nxt_dma = start_dma(step_i + 1) if step_i + 1 < len(steps) else None
            pending.wait()
            buf = bufs[step_i % 2]
            lab_base = s * LPAD + c * DCHUNK

            def body(i, a, buf=buf, lab_base=lab_base):
                tok = i * LANES
                prev = lab_v[pl.ds(lab_base + tok, LANES)]
                nxt = plsc.load_gather(lab_v, [lab_base + tok + 1 + iota])
                tv = plsc.load_gather(
                    tab_v, [(prev - 2) * (L + 1) + (nxt - 2)])
                ev = plsc.load_gather(buf, [(tok + iota) * C + prev])
                return a + tv + ev

            acc = lax.fori_loop(0, DCHUNK // LANES, body, acc)
            pending = nxt_dma

        for s in range(seq_per_w):
            first = lab_v[pl.ds(s * LPAD, LANES)]
            sv = plsc.load_gather(tab_v, [L * (L + 1) + (first - 2)])
            acc = acc + jnp.where(iota == 0, sv, jnp.zeros_like(sv))

        acc_v[...] = acc
        pltpu.sync_copy(acc_v, out_hbm.at[pl.ds(wid * LANES, LANES)])

    return sc_call


def kernel(log_probs, input_lens, labels, A_scores):
    BS, T, C = log_probs.shape
    log_pi = jax.nn.log_softmax(A_scores[:L])
    log_trans = jax.nn.log_softmax(A_scores[L:].reshape(L, L + 1), axis=-1)
    table = jnp.concatenate(
        [log_trans.reshape(-1), log_pi,
         jnp.zeros((TABLE_PAD - L - L * (L + 1),), jnp.float32)])

    info = plsc.get_sparse_core_info()
    n_workers = info.num_cores * info.num_subcores
    sc_call = _make_sc_call(BS, T, C, n_workers)
    partials = sc_call(log_probs.reshape(-1), labels.reshape(-1), table)
    return jnp.sum(partials) / (BS * T)


# speedup vs baseline: 124.0123x; 6.2805x over previous
import functools

import jax
import jax.numpy as jnp
from jax import lax
from jax.experimental import pallas as pl
from jax.experimental.pallas import tpu as pltpu
from jax.experimental.pallas import tpu_sc as plsc

L = 16
NCLASS = 18
LANES = 16
TABLE_PAD = 384


def _make_sc_call(BS, T, n_workers):
    seq_per_w = BS // n_workers
    KT = T // 128
    P = BS * T
    n_blocks = seq_per_w * KT
    LAB_N = KT * 256
    mesh = plsc.VectorSubcoreMesh(core_axis_name="c", subcore_axis_name="s")

    @functools.partial(
        pl.kernel,
        out_type=jax.ShapeDtypeStruct((n_workers * LANES,), jnp.float32),
        mesh=mesh,
        scratch_types=[
            pltpu.VMEM((LAB_N + 256,), jnp.int32),
            pltpu.VMEM((n_blocks, 128), jnp.int32),
            pltpu.VMEM((n_blocks * 128,), jnp.float32),
            pltpu.VMEM((TABLE_PAD,), jnp.float32),
            pltpu.VMEM((LANES,), jnp.float32),
            pltpu.SemaphoreType.DMA,
            pltpu.SemaphoreType.DMA,
        ],
        compiler_params=pltpu.CompilerParams(needs_layout_passes=False),
    )
    def sc_call(lp_hbm, lab_hbm, tab_hbm, out_hbm,
                lab_v, idx_v, val_v, tab_v, acc_v, sem_lab, sem_em):
        nc = mesh.num_cores
        wid = lax.axis_index("s") * nc + lax.axis_index("c")
        iota = lax.iota(jnp.int32, LANES)
        g = wid // 4
        r0 = (wid * seq_per_w) % 8

        pltpu.sync_copy(tab_hbm, tab_v)
        lab_src_base = g * (8 * T) + r0 * 128
        for k in range(KT):
            pltpu.async_copy(
                lab_hbm.at[pl.ds(lab_src_base + k * 1024, 256)],
                lab_v.at[pl.ds(k * 256, 256)], sem_lab)
        for j in range(seq_per_w):
            lab_v[pl.ds(LAB_N + j * 128, LANES)] = jnp.full(
                (LANES,), NCLASS, jnp.int32)
        pltpu.make_async_copy(
            lab_hbm.at[pl.ds(0, LAB_N)], lab_v.at[pl.ds(0, LAB_N)],
            sem_lab).wait()

        acc = jnp.zeros((LANES,), jnp.float32)
        for j in range(seq_per_w):
            e_base = g * (8 * T) + (r0 + j) * 128

            def body(k, a, j=j, e_base=e_base):
                row = j * KT + k
                for i in range(8):
                    off = i * LANES
                    prev = lab_v[pl.ds(k * 256 + j * 128 + off, LANES)]
                    tt = k * 128 + off + 1 + iota
                    nxt = plsc.load_gather(
                        lab_v,
                        [(tt >> 7) * 256 + j * 128 + (tt & 127)])
                    tv = plsc.load_gather(
                        tab_v, [prev * (L + 1) + nxt - (2 * (L + 1) + 2)])
                    idx_v[row, pl.ds(off, LANES)] = (
                        prev * P + (e_base + k * 1024 + off + iota))
                    a = a + tv
                pltpu.async_copy(
                    lp_hbm.at[idx_v.at[row]],
                    val_v.at[pl.ds(row * 128, 128)], sem_em)
                return a

            acc = lax.fori_loop(0, KT, body, acc)

            first = lab_v[pl.ds(j * 128, LANES)]
            sv = plsc.load_gather(tab_v, [L * (L + 1) + (first - 2)])
            acc = acc + jnp.where(iota == 0, sv, jnp.zeros_like(sv))

        pltpu.make_async_copy(
            lp_hbm.at[pl.ds(0, n_blocks * 128)], val_v, sem_em).wait()

        def red(n, a):
            return a + val_v[pl.ds(n * LANES, LANES)]
        acc = lax.fori_loop(0, n_blocks * 128 // LANES, red, acc)

        acc_v[...] = acc
        pltpu.sync_copy(acc_v, out_hbm.at[pl.ds(wid * LANES, LANES)])

    return sc_call


def kernel(log_probs, input_lens, labels, A_scores):
    BS, T, C = log_probs.shape
    log_pi = jax.nn.log_softmax(A_scores[:L])
    log_trans = jax.nn.log_softmax(A_scores[L:].reshape(L, L + 1), axis=-1)
    table = jnp.concatenate(
        [log_trans.reshape(-1), log_pi,
         jnp.zeros((TABLE_PAD - L - L * (L + 1),), jnp.float32)])

    lp_view = (log_probs
               .transpose(2, 0, 1)
               .reshape(C, BS // 8, 8, T // 128, 128)
               .transpose(0, 1, 3, 2, 4)
               .reshape(-1))
    lab_view = (labels
                .reshape(BS // 8, 8, T // 128, 128)
                .transpose(0, 2, 1, 3)
                .reshape(-1))

    info = plsc.get_sparse_core_info()
    n_workers = info.num_cores * info.num_subcores
    sc_call = _make_sc_call(BS, T, n_workers)
    partials = sc_call(lp_view, lab_view, table)
    return jnp.sum(partials) / (BS * T)


# speedup vs baseline: 132.6494x; 1.0696x over previous
import functools

import jax
import jax.numpy as jnp
from jax import lax
from jax.experimental import pallas as pl
from jax.experimental.pallas import tpu as pltpu
from jax.experimental.pallas import tpu_sc as plsc

L = 16
NCLASS = 18
LANES = 16
TABLE_PAD = 384


def _make_sc_call(BS, T, n_workers):
    seq_per_w = BS // n_workers
    KT = T // 128
    P = BS * T
    n_blocks = seq_per_w * KT
    LAB_N = KT * 256
    mesh = plsc.VectorSubcoreMesh(core_axis_name="c", subcore_axis_name="s")

    @functools.partial(
        pl.kernel,
        out_type=jax.ShapeDtypeStruct((n_workers * LANES,), jnp.float32),
        mesh=mesh,
        scratch_types=[
            pltpu.VMEM((LAB_N + 256,), jnp.int32),
            pltpu.VMEM((n_blocks, 128), jnp.int32),
            pltpu.VMEM((n_blocks * 128,), jnp.float32),
            pltpu.VMEM((TABLE_PAD,), jnp.float32),
            pltpu.VMEM((LANES,), jnp.float32),
            pltpu.SemaphoreType.DMA,
            pltpu.SemaphoreType.DMA,
        ],
        compiler_params=pltpu.CompilerParams(needs_layout_passes=False),
    )
    def sc_call(lp_hbm, lab_hbm, tab_hbm, out_hbm,
                lab_v, idx_v, val_v, tab_v, acc_v, sem_lab, sem_em):
        nc = mesh.num_cores
        wid = lax.axis_index("s") * nc + lax.axis_index("c")
        iota = lax.iota(jnp.int32, LANES)
        g = wid // 4
        r0 = (wid * seq_per_w) % 8

        pltpu.sync_copy(tab_hbm, tab_v)
        lab_src_base = g * (8 * T) + r0 * 128
        for k in range(KT):
            pltpu.async_copy(
                lab_hbm.at[pl.ds(lab_src_base + k * 1024, 256)],
                lab_v.at[pl.ds(k * 256, 256)], sem_lab)
        for j in range(seq_per_w):
            lab_v[pl.ds(LAB_N + j * 128, LANES)] = jnp.full(
                (LANES,), NCLASS, jnp.int32)
        pltpu.make_async_copy(
            lab_hbm.at[pl.ds(0, LAB_N)], lab_v.at[pl.ds(0, LAB_N)],
            sem_lab).wait()

        acc = jnp.zeros((LANES,), jnp.float32)
        zero = jnp.zeros((LANES,), jnp.float32)
        for j in range(seq_per_w):
            e_base = g * (8 * T) + (r0 + j) * 128

            def body(k, carry, j=j, e_base=e_base):
                a0, a1 = carry
                row = j * KT + k
                lbase = k * 256 + j * 128
                for i in range(8):
                    off = i * LANES
                    prev = lab_v[pl.ds(lbase + off, LANES)]
                    if i < 7:
                        nxt = lab_v[pl.ds(lbase + off + 1, LANES)]
                    else:
                        tt = k * 128 + off + 1 + iota
                        nxt = plsc.load_gather(
                            lab_v,
                            [(tt >> 7) * 256 + j * 128 + (tt & 127)])
                    tv = plsc.load_gather(
                        tab_v, [prev * (L + 1) + nxt - (2 * (L + 1) + 2)])
                    idx_v[row, pl.ds(off, LANES)] = (
                        prev * P + (e_base + k * 1024 + off + iota))
                    if i % 2 == 0:
                        a0 = a0 + tv
                    else:
                        a1 = a1 + tv
                pltpu.async_copy(
                    lp_hbm.at[idx_v.at[row]],
                    val_v.at[pl.ds(row * 128, 128)], sem_em)
                return a0, a1

            acc, acc1 = lax.fori_loop(0, KT, body, (acc, zero))
            acc = acc + acc1

            first = lab_v[pl.ds(j * 128, LANES)]
            sv = plsc.load_gather(tab_v, [L * (L + 1) + (first - 2)])
            acc = acc + jnp.where(iota == 0, sv, jnp.zeros_like(sv))

        pltpu.make_async_copy(
            lp_hbm.at[pl.ds(0, n_blocks * 128)], val_v, sem_em).wait()

        def red(n, c):
            b0, b1, b2, b3 = c
            rb = n * (4 * LANES)
            return (b0 + val_v[pl.ds(rb, LANES)],
                    b1 + val_v[pl.ds(rb + LANES, LANES)],
                    b2 + val_v[pl.ds(rb + 2 * LANES, LANES)],
                    b3 + val_v[pl.ds(rb + 3 * LANES, LANES)])
        b0, b1, b2, b3 = lax.fori_loop(
            0, n_blocks * 128 // (4 * LANES), red, (acc, zero, zero, zero))
        acc = (b0 + b1) + (b2 + b3)

        acc_v[...] = acc
        pltpu.sync_copy(acc_v, out_hbm.at[pl.ds(wid * LANES, LANES)])

    return sc_call


def kernel(log_probs, input_lens, labels, A_scores):
    BS, T, C = log_probs.shape
    log_pi = jax.nn.log_softmax(A_scores[:L])
    log_trans = jax.nn.log_softmax(A_scores[L:].reshape(L, L + 1), axis=-1)
    table = jnp.concatenate(
        [log_trans.reshape(-1), log_pi,
         jnp.zeros((TABLE_PAD - L - L * (L + 1),), jnp.float32)])

    lp_view = (log_probs
               .transpose(2, 0, 1)
               .reshape(C, BS // 8, 8, T // 128, 128)
               .transpose(0, 1, 3, 2, 4)
               .reshape(-1))
    lab_view = (labels
                .reshape(BS // 8, 8, T // 128, 128)
                .transpose(0, 2, 1, 3)
                .reshape(-1))

    info = plsc.get_sparse_core_info()
    n_workers = info.num_cores * info.num_subcores
    sc_call = _make_sc_call(BS, T, n_workers)
    partials = sc_call(lp_view, lab_view, table)
    return jnp.sum(partials) / (BS * T)


# speedup vs baseline: 147.4897x; 1.1119x over previous
import functools

import jax
import jax.numpy as jnp
from jax import lax
from jax.experimental import pallas as pl
from jax.experimental.pallas import tpu as pltpu
from jax.experimental.pallas import tpu_sc as plsc

L = 16
NCLASS = 18
LANES = 16
TABLE_PAD = 384
NPARAM = L + L * (L + 1)
SCRATCH0 = 352


def _log_sc(s):
    bits = plsc.bitcast(s, jnp.int32)
    y = (bits.astype(jnp.float32) - 1064866805.0) * 8.262958405176314e-08
    for _ in range(3):
        y = y - 1.0 + s * jnp.exp(-y)
    return y


def _make_sc_call(BS, T, n_workers):
    seq_per_w = BS // n_workers
    KT = T // 128
    P = BS * T
    n_blocks = seq_per_w * KT
    LAB_N = KT * 256
    mesh = plsc.VectorSubcoreMesh(core_axis_name="c", subcore_axis_name="s")

    @functools.partial(
        pl.kernel,
        out_type=jax.ShapeDtypeStruct((n_workers * LANES,), jnp.float32),
        mesh=mesh,
        scratch_types=[
            pltpu.VMEM((LAB_N + 256,), jnp.int32),
            pltpu.VMEM((n_blocks, 128), jnp.int32),
            pltpu.VMEM((n_blocks * 128,), jnp.float32),
            pltpu.VMEM((TABLE_PAD,), jnp.float32),
            pltpu.VMEM((NPARAM,), jnp.float32),
            pltpu.VMEM((LANES,), jnp.float32),
            pltpu.SemaphoreType.DMA,
            pltpu.SemaphoreType.DMA,
        ],
        compiler_params=pltpu.CompilerParams(needs_layout_passes=False),
    )
    def sc_call(lp_hbm, lab_hbm, asc_hbm, out_hbm,
                lab_v, idx_v, val_v, tab_v, raw_v, acc_v, sem_lab, sem_em):
        nc = mesh.num_cores
        wid = lax.axis_index("s") * nc + lax.axis_index("c")
        iota = lax.iota(jnp.int32, LANES)
        g = wid // 4
        r0 = (wid * seq_per_w) % 8

        pltpu.sync_copy(asc_hbm, raw_v)
        cols = [plsc.load_gather(raw_v, [iota * (L + 1) + (L + j)])
                for j in range(L + 1)]
        m = cols[0]
        for j in range(1, L + 1):
            m = jnp.maximum(m, cols[j])
        s = jnp.exp(cols[0] - m)
        for j in range(1, L + 1):
            s = s + jnp.exp(cols[j] - m)
        lse = m + _log_sc(s)
        for j in range(L + 1):
            plsc.store_scatter(tab_v, [iota * (L + 1) + j], cols[j] - lse)
        v0 = raw_v[pl.ds(0, LANES)]
        tab_v[pl.ds(SCRATCH0, LANES)] = plsc.cummax(v0)
        m0 = plsc.load_gather(
            tab_v, [jnp.full((LANES,), SCRATCH0 + LANES - 1, jnp.int32)])
        e0 = jnp.exp(v0 - m0)
        tab_v[pl.ds(SCRATCH0, LANES)] = plsc.cumsum(e0)
        s0 = plsc.load_gather(
            tab_v, [jnp.full((LANES,), SCRATCH0 + LANES - 1, jnp.int32)])
        tab_v[pl.ds(L * (L + 1), LANES)] = v0 - m0 - _log_sc(s0)
        lab_src_base = g * (8 * T) + r0 * 128
        for k in range(KT):
            pltpu.async_copy(
                lab_hbm.at[pl.ds(lab_src_base + k * 1024, 256)],
                lab_v.at[pl.ds(k * 256, 256)], sem_lab)
        for j in range(seq_per_w):
            lab_v[pl.ds(LAB_N + j * 128, LANES)] = jnp.full(
                (LANES,), NCLASS, jnp.int32)
        pltpu.make_async_copy(
            lab_hbm.at[pl.ds(0, LAB_N)], lab_v.at[pl.ds(0, LAB_N)],
            sem_lab).wait()

        acc = jnp.zeros((LANES,), jnp.float32)
        zero = jnp.zeros((LANES,), jnp.float32)
        for j in range(seq_per_w):
            e_base = g * (8 * T) + (r0 + j) * 128

            def body(k, carry, j=j, e_base=e_base):
                a0, a1 = carry
                row = j * KT + k
                lbase = k * 256 + j * 128
                for i in range(8):
                    off = i * LANES
                    prev = lab_v[pl.ds(lbase + off, LANES)]
                    if i < 7:
                        nxt = lab_v[pl.ds(lbase + off + 1, LANES)]
                    else:
                        tt = k * 128 + off + 1 + iota
                        nxt = plsc.load_gather(
                            lab_v,
                            [(tt >> 7) * 256 + j * 128 + (tt & 127)])
                    tv = plsc.load_gather(
                        tab_v, [prev * (L + 1) + nxt - (2 * (L + 1) + 2)])
                    idx_v[row, pl.ds(off, LANES)] = (
                        prev * P + (e_base + k * 1024 + off + iota))
                    if i % 2 == 0:
                        a0 = a0 + tv
                    else:
                        a1 = a1 + tv
                pltpu.async_copy(
                    lp_hbm.at[idx_v.at[row]],
                    val_v.at[pl.ds(row * 128, 128)], sem_em)
                return a0, a1

            acc, acc1 = lax.fori_loop(0, KT, body, (acc, zero))
            acc = acc + acc1

            first = lab_v[pl.ds(j * 128, LANES)]
            sv = plsc.load_gather(tab_v, [L * (L + 1) + (first - 2)])
            acc = acc + jnp.where(iota == 0, sv, jnp.zeros_like(sv))

        pltpu.make_async_copy(
            lp_hbm.at[pl.ds(0, n_blocks * 128)], val_v, sem_em).wait()

        def red(n, c):
            b0, b1, b2, b3 = c
            rb = n * (4 * LANES)
            return (b0 + val_v[pl.ds(rb, LANES)],
                    b1 + val_v[pl.ds(rb + LANES, LANES)],
                    b2 + val_v[pl.ds(rb + 2 * LANES, LANES)],
                    b3 + val_v[pl.ds(rb + 3 * LANES, LANES)])
        b0, b1, b2, b3 = lax.fori_loop(
            0, n_blocks * 128 // (4 * LANES), red, (acc, zero, zero, zero))
        acc = (b0 + b1) + (b2 + b3)

        acc_v[...] = acc
        pltpu.sync_copy(acc_v, out_hbm.at[pl.ds(wid * LANES, LANES)])

    return sc_call


def kernel(log_probs, input_lens, labels, A_scores):
    BS, T, C = log_probs.shape
    lp_view = (log_probs
               .transpose(2, 0, 1)
               .reshape(C, BS // 8, 8, T // 128, 128)
               .transpose(0, 1, 3, 2, 4)
               .reshape(-1))
    lab_view = (labels
                .reshape(BS // 8, 8, T // 128, 128)
                .transpose(0, 2, 1, 3)
                .reshape(-1))

    info = plsc.get_sparse_core_info()
    n_workers = info.num_cores * info.num_subcores
    sc_call = _make_sc_call(BS, T, n_workers)
    partials = sc_call(lp_view, lab_view, A_scores)
    return jnp.sum(partials) / (BS * T)
